# baseline (speedup 1.0000x reference)
import functools

import jax
import jax.numpy as jnp
from jax import lax
from jax.experimental import pallas as pl
from jax.experimental.pallas import tpu as pltpu
from jax.experimental.pallas import tpu_sc as plsc

_T_INV = 20.0
_EPS = 1e-12
_M = 100000
_D = 128
_B = 1024
_BM = 2048
_GRID = (_M + _BM - 1) // _BM

_NC = 2
_NS = 16
_NW = _NC * _NS
_BPW = _B // _NW


def _tc_body(x_ref, idxc_ref, idxr_ref, mem_ref,
             out_ref, draft_ref, feat_ref, win_ref, feat_s):
    i = pl.program_id(0)

    @pl.when(i == 0)
    def _prologue():
        xv = x_ref[...]
        norm = jnp.sqrt(jnp.sum(xv * xv, axis=1, keepdims=True))
        feat = xv / (norm + _EPS)
        feat_s[...] = feat.astype(jnp.bfloat16)
        feat_ref[...] = feat
        eq = idxc_ref[...] == idxr_ref[...]
        pos = lax.broadcasted_iota(jnp.int32, (_B, _B), 1)
        win_ref[...] = jnp.max(jnp.where(eq, pos, -1), axis=1, keepdims=True)

    mem = mem_ref[...]
    out_ref[...] = lax.dot_general(
        mem.astype(jnp.bfloat16), feat_s[...], (((1,), (1,)), ((), ())),
        preferred_element_type=jnp.float32) * _T_INV
    draft_ref[...] = mem


def _tc_call(x, idxc, idxr, memory):
    return pl.pallas_call(
        _tc_body,
        grid=(_GRID,),
        in_specs=[
            pl.BlockSpec((_B, _D), lambda i: (0, 0)),
            pl.BlockSpec((_B, 1), lambda i: (0, 0)),
            pl.BlockSpec((1, _B), lambda i: (0, 0)),
            pl.BlockSpec((_BM, _D), lambda i: (i, 0)),
        ],
        out_specs=[
            pl.BlockSpec((_BM, _B), lambda i: (i, 0)),
            pl.BlockSpec((_BM, _D), lambda i: (i, 0)),
            pl.BlockSpec((_B, _D), lambda i: (0, 0)),
            pl.BlockSpec((_B, 1), lambda i: (0, 0)),
        ],
        out_shape=[
            jax.ShapeDtypeStruct((_M, _B), jnp.float32),
            jax.ShapeDtypeStruct((_M, _D), jnp.float32),
            jax.ShapeDtypeStruct((_B, _D), jnp.float32),
            jax.ShapeDtypeStruct((_B, 1), jnp.int32),
        ],
        scratch_shapes=[pltpu.VMEM((_B, _D), jnp.bfloat16)],
        compiler_params=pltpu.CompilerParams(
            dimension_semantics=("arbitrary",)),
    )(x, idxc, idxr, memory)


def _sc_scatter_body(idx_hbm, win_hbm, feat_hbm, draft_ref,
                     idx_v, win_v, rows_v, sem):
    wid = lax.axis_index("s") * _NC + lax.axis_index("c")
    base = wid * _BPW
    pltpu.sync_copy(idx_hbm.at[pl.ds(base, _BPW)], idx_v)
    pltpu.sync_copy(win_hbm.at[pl.ds(base, _BPW)], win_v)
    pltpu.async_copy(feat_hbm.at[win_v], rows_v, sem).wait()
    pltpu.async_copy(rows_v, draft_ref.at[idx_v], sem).wait()


@functools.cache
def _sc_scatter():
    return functools.partial(
        pl.kernel,
        out_type=(),
        mesh=plsc.VectorSubcoreMesh(
            core_axis_name="c", subcore_axis_name="s",
            num_cores=_NC, num_subcores=_NS),
        scratch_types=[
            pltpu.VMEM((_BPW,), jnp.int32),
            pltpu.VMEM((_BPW,), jnp.int32),
            pltpu.VMEM((_BPW, _D), jnp.float32),
            pltpu.SemaphoreType.DMA,
        ],
    )(_sc_scatter_body)


def kernel(x, index, memory):
    idxc = index.reshape(_B, 1)
    idxr = index.reshape(1, _B)
    out_t, draft, feat, win = _tc_call(x, idxc, idxr, memory)
    draft_ref = jax.new_ref(draft)
    _sc_scatter()(index, win.reshape(_B), feat, draft_ref)
    return out_t.T, jax.freeze(draft_ref)


# speedup vs baseline: 1.0143x; 1.0143x over previous
import functools

import jax
import jax.numpy as jnp
from jax import lax
from jax.experimental import pallas as pl
from jax.experimental.pallas import tpu as pltpu
from jax.experimental.pallas import tpu_sc as plsc

_T_INV = 20.0
_EPS = 1e-12
_M = 100000
_D = 128
_B = 1024
_BM = 4096
_GRID = (_M + _BM - 1) // _BM

_NC = 2
_NS = 16
_NW = _NC * _NS
_BPW = _B // _NW


def _tc_body(x_ref, idxc_ref, idxr_ref, mem_ref,
             out_ref, draft_ref, feat_ref, win_ref, feat_s):
    i = pl.program_id(0)

    @pl.when(i == 0)
    def _prologue():
        xv = x_ref[...]
        norm = jnp.sqrt(jnp.sum(xv * xv, axis=1, keepdims=True))
        feat = xv / (norm + _EPS)
        feat_s[...] = feat.astype(jnp.bfloat16)
        feat_ref[...] = feat
        eq = idxc_ref[...] == idxr_ref[...]
        pos = lax.broadcasted_iota(jnp.int32, (_B, _B), 1)
        win_ref[...] = jnp.max(jnp.where(eq, pos, -1), axis=1, keepdims=True)

    mem = mem_ref[...]
    out_ref[...] = lax.dot_general(
        mem.astype(jnp.bfloat16), feat_s[...], (((1,), (1,)), ((), ())),
        preferred_element_type=jnp.float32) * _T_INV
    draft_ref[...] = mem


def _tc_call(x, idxc, idxr, memory):
    return pl.pallas_call(
        _tc_body,
        grid=(_GRID,),
        in_specs=[
            pl.BlockSpec((_B, _D), lambda i: (0, 0)),
            pl.BlockSpec((_B, 1), lambda i: (0, 0)),
            pl.BlockSpec((1, _B), lambda i: (0, 0)),
            pl.BlockSpec((_BM, _D), lambda i: (i, 0)),
        ],
        out_specs=[
            pl.BlockSpec((_BM, _B), lambda i: (i, 0)),
            pl.BlockSpec((_BM, _D), lambda i: (i, 0)),
            pl.BlockSpec((_B, _D), lambda i: (0, 0)),
            pl.BlockSpec((_B, 1), lambda i: (0, 0)),
        ],
        out_shape=[
            jax.ShapeDtypeStruct((_M, _B), jnp.float32),
            jax.ShapeDtypeStruct((_M, _D), jnp.float32),
            jax.ShapeDtypeStruct((_B, _D), jnp.float32),
            jax.ShapeDtypeStruct((_B, 1), jnp.int32),
        ],
        scratch_shapes=[pltpu.VMEM((_B, _D), jnp.bfloat16)],
        compiler_params=pltpu.CompilerParams(
            dimension_semantics=("arbitrary",)),
    )(x, idxc, idxr, memory)


def _sc_scatter_body(idx_hbm, win_hbm, feat_hbm, draft_ref,
                     idx_v, win_v, rows_v, sem):
    wid = lax.axis_index("s") * _NC + lax.axis_index("c")
    base = wid * _BPW
    pltpu.sync_copy(idx_hbm.at[pl.ds(base, _BPW)], idx_v)
    pltpu.sync_copy(win_hbm.at[pl.ds(base, _BPW)], win_v)
    pltpu.async_copy(feat_hbm.at[win_v], rows_v, sem).wait()
    pltpu.async_copy(rows_v, draft_ref.at[idx_v], sem).wait()


@functools.cache
def _sc_scatter():
    return functools.partial(
        pl.kernel,
        out_type=(),
        mesh=plsc.VectorSubcoreMesh(
            core_axis_name="c", subcore_axis_name="s",
            num_cores=_NC, num_subcores=_NS),
        scratch_types=[
            pltpu.VMEM((_BPW,), jnp.int32),
            pltpu.VMEM((_BPW,), jnp.int32),
            pltpu.VMEM((_BPW, _D), jnp.float32),
            pltpu.SemaphoreType.DMA,
        ],
    )(_sc_scatter_body)


def kernel(x, index, memory):
    idxc = index.reshape(_B, 1)
    idxr = index.reshape(1, _B)
    out_t, draft, feat, win = _tc_call(x, idxc, idxr, memory)
    draft_ref = jax.new_ref(draft)
    _sc_scatter()(index, win.reshape(_B), feat, draft_ref)
    return out_t.T, jax.freeze(draft_ref)


# speedup vs baseline: 1.0153x; 1.0009x over previous
import functools

import jax
import jax.numpy as jnp
from jax import lax
from jax.experimental import pallas as pl
from jax.experimental.pallas import tpu as pltpu
from jax.experimental.pallas import tpu_sc as plsc

_T_INV = 20.0
_EPS = 1e-12
_M = 100000
_D = 128
_B = 1024
_BM = 5120
_GRID = (_M + _BM - 1) // _BM

_NC = 2
_NS = 16
_NW = _NC * _NS
_BPW = _B // _NW


def _tc_body(x_ref, idxc_ref, idxr_ref, mem_ref,
             out_ref, draft_ref, feat_ref, win_ref, feat_s):
    i = pl.program_id(0)

    @pl.when(i == 0)
    def _prologue():
        xv = x_ref[...]
        norm = jnp.sqrt(jnp.sum(xv * xv, axis=1, keepdims=True))
        feat = xv / (norm + _EPS)
        feat_s[...] = feat.astype(jnp.bfloat16)
        feat_ref[...] = feat
        eq = idxc_ref[...] == idxr_ref[...]
        pos = lax.broadcasted_iota(jnp.int32, (_B, _B), 1)
        win_ref[...] = jnp.max(jnp.where(eq, pos, -1), axis=1, keepdims=True)

    mem = mem_ref[...]
    out_ref[...] = lax.dot_general(
        mem.astype(jnp.bfloat16), feat_s[...], (((1,), (1,)), ((), ())),
        preferred_element_type=jnp.float32) * _T_INV
    draft_ref[...] = mem


def _tc_call(x, idxc, idxr, memory):
    return pl.pallas_call(
        _tc_body,
        grid=(_GRID,),
        in_specs=[
            pl.BlockSpec((_B, _D), lambda i: (0, 0)),
            pl.BlockSpec((_B, 1), lambda i: (0, 0)),
            pl.BlockSpec((1, _B), lambda i: (0, 0)),
            pl.BlockSpec((_BM, _D), lambda i: (i, 0)),
        ],
        out_specs=[
            pl.BlockSpec((_BM, _B), lambda i: (i, 0)),
            pl.BlockSpec((_BM, _D), lambda i: (i, 0)),
            pl.BlockSpec((_B, _D), lambda i: (0, 0)),
            pl.BlockSpec((_B, 1), lambda i: (0, 0)),
        ],
        out_shape=[
            jax.ShapeDtypeStruct((_M, _B), jnp.float32),
            jax.ShapeDtypeStruct((_M, _D), jnp.float32),
            jax.ShapeDtypeStruct((_B, _D), jnp.float32),
            jax.ShapeDtypeStruct((_B, 1), jnp.int32),
        ],
        scratch_shapes=[pltpu.VMEM((_B, _D), jnp.bfloat16)],
        compiler_params=pltpu.CompilerParams(
            dimension_semantics=("arbitrary",)),
    )(x, idxc, idxr, memory)


def _sc_scatter_body(idx_hbm, win_hbm, feat_hbm, draft_ref,
                     idx_v, win_v, rows_v, sem):
    wid = lax.axis_index("s") * _NC + lax.axis_index("c")
    base = wid * _BPW
    pltpu.sync_copy(idx_hbm.at[pl.ds(base, _BPW)], idx_v)
    pltpu.sync_copy(win_hbm.at[pl.ds(base, _BPW)], win_v)
    pltpu.async_copy(feat_hbm.at[win_v], rows_v, sem).wait()
    pltpu.async_copy(rows_v, draft_ref.at[idx_v], sem).wait()


@functools.cache
def _sc_scatter():
    return functools.partial(
        pl.kernel,
        out_type=(),
        mesh=plsc.VectorSubcoreMesh(
            core_axis_name="c", subcore_axis_name="s",
            num_cores=_NC, num_subcores=_NS),
        scratch_types=[
            pltpu.VMEM((_BPW,), jnp.int32),
            pltpu.VMEM((_BPW,), jnp.int32),
            pltpu.VMEM((_BPW, _D), jnp.float32),
            pltpu.SemaphoreType.DMA,
        ],
    )(_sc_scatter_body)


def kernel(x, index, memory):
    idxc = index.reshape(_B, 1)
    idxr = index.reshape(1, _B)
    out_t, draft, feat, win = _tc_call(x, idxc, idxr, memory)
    draft_ref = jax.new_ref(draft)
    _sc_scatter()(index, win.reshape(_B), feat, draft_ref)
    return out_t.T, jax.freeze(draft_ref)


# speedup vs baseline: 1.0190x; 1.0036x over previous
import functools

import jax
import jax.numpy as jnp
from jax import lax
from jax.experimental import pallas as pl
from jax.experimental.pallas import tpu as pltpu
from jax.experimental.pallas import tpu_sc as plsc

_T_INV = 20.0
_EPS = 1e-12
_M = 100000
_D = 128
_B = 1024
_BM = 4096
_GRID = (_M + _BM - 1) // _BM

_NC = 2
_NS = 16
_NW = _NC * _NS
_BPW = _B // _NW


def _tc_body(x_ref, idxc_ref, idxr_ref, mem_ref,
             out_ref, draft_ref, feat_ref, win_ref, feat_s):
    i = pl.program_id(0)

    @pl.when(i == 0)
    def _prologue():
        xv = x_ref[...]
        norm = jnp.sqrt(jnp.sum(xv * xv, axis=1, keepdims=True))
        feat = xv / (norm + _EPS)
        feat_s[...] = feat.astype(jnp.bfloat16)
        feat_ref[...] = feat

    @pl.when(i == _GRID - 1)
    def _epilogue():
        eq = idxc_ref[...] == idxr_ref[...]
        pos = lax.broadcasted_iota(jnp.int32, (_B, _B), 1)
        win_ref[...] = jnp.max(jnp.where(eq, pos, -1), axis=1, keepdims=True)

    mem = mem_ref[...]
    out_ref[...] = lax.dot_general(
        mem.astype(jnp.bfloat16), feat_s[...], (((1,), (1,)), ((), ())),
        preferred_element_type=jnp.float32) * _T_INV
    draft_ref[...] = mem


def _tc_call(x, idxc, idxr, memory):
    return pl.pallas_call(
        _tc_body,
        grid=(_GRID,),
        in_specs=[
            pl.BlockSpec((_B, _D), lambda i: (0, 0)),
            pl.BlockSpec((_B, 1), lambda i: (0, 0)),
            pl.BlockSpec((1, _B), lambda i: (0, 0)),
            pl.BlockSpec((_BM, _D), lambda i: (i, 0)),
        ],
        out_specs=[
            pl.BlockSpec((_BM, _B), lambda i: (i, 0)),
            pl.BlockSpec((_BM, _D), lambda i: (i, 0)),
            pl.BlockSpec((_B, _D), lambda i: (0, 0)),
            pl.BlockSpec((_B, 1), lambda i: (0, 0)),
        ],
        out_shape=[
            jax.ShapeDtypeStruct((_M, _B), jnp.float32),
            jax.ShapeDtypeStruct((_M, _D), jnp.float32),
            jax.ShapeDtypeStruct((_B, _D), jnp.float32),
            jax.ShapeDtypeStruct((_B, 1), jnp.int32),
        ],
        scratch_shapes=[pltpu.VMEM((_B, _D), jnp.bfloat16)],
        compiler_params=pltpu.CompilerParams(
            dimension_semantics=("arbitrary",)),
    )(x, idxc, idxr, memory)


def _sc_scatter_body(idx_hbm, win_hbm, feat_hbm, draft_ref,
                     idx_v, win_v, rows_v, sem):
    wid = lax.axis_index("s") * _NC + lax.axis_index("c")
    base = wid * _BPW
    pltpu.sync_copy(idx_hbm.at[pl.ds(base, _BPW)], idx_v)
    pltpu.sync_copy(win_hbm.at[pl.ds(base, _BPW)], win_v)
    pltpu.async_copy(feat_hbm.at[win_v], rows_v, sem).wait()
    pltpu.async_copy(rows_v, draft_ref.at[idx_v], sem).wait()


@functools.cache
def _sc_scatter():
    return functools.partial(
        pl.kernel,
        out_type=(),
        mesh=plsc.VectorSubcoreMesh(
            core_axis_name="c", subcore_axis_name="s",
            num_cores=_NC, num_subcores=_NS),
        scratch_types=[
            pltpu.VMEM((_BPW,), jnp.int32),
            pltpu.VMEM((_BPW,), jnp.int32),
            pltpu.VMEM((_BPW, _D), jnp.float32),
            pltpu.SemaphoreType.DMA,
        ],
    )(_sc_scatter_body)


def kernel(x, index, memory):
    idxc = index.reshape(_B, 1)
    idxr = index.reshape(1, _B)
    out_t, draft, feat, win = _tc_call(x, idxc, idxr, memory)
    draft_ref = jax.new_ref(draft)
    _sc_scatter()(index, win.reshape(_B), feat, draft_ref)
    return out_t.T, jax.freeze(draft_ref)
